# baseline (speedup 1.0000x reference)
import jax
import jax.numpy as jnp
from jax.experimental import pallas as pl
from jax.experimental.pallas import tpu as pltpu

B, N, IN_DIM, OUT_DIM, HEADS = 4, 512, 128, 128, 4
C = OUT_DIM // HEADS
NEG_SLOPE = 0.2
TJ = 128
NJ = N // TJ


def _gat_tile_kernel(x_ref, adj_ref, wl_ref, bl_ref, wr_ref, br_ref,
                     att_ref, att_col_ref, bias_ref, gamma_ref, beta_ref,
                     out_ref):
    jt = pl.program_id(1)
    xb = x_ref[0]
    xl = jnp.dot(xb, wl_ref[...], preferred_element_type=jnp.float32)
    xl = xl + bl_ref[0]
    xbt = x_ref[0, pl.ds(jt * TJ, TJ), :]
    xrt = jax.lax.dot_general(wr_ref[...], xbt, (((0,), (1,)), ((), ())),
                              preferred_element_type=jnp.float32)
    mask = adj_ref[0] != 0
    neg_inf = jnp.float32(-jnp.inf)
    ones_col = jnp.ones((N, 1), jnp.float32)
    head_outs = []
    for h in range(HEADS):
        xl_h = xl[:, h * C:(h + 1) * C]
        u = jnp.dot(xl_h, att_col_ref[h * C:(h + 1) * C, :],
                    preferred_element_type=jnp.float32)
        q = u * jnp.float32(0.6)
        for c in range(C):
            hc = h * C + c
            col = xl_h[:, c:c + 1] + br_ref[0, hc]
            z = col + xrt[hc:hc + 1, :]
            q = q + jnp.abs(z) * (att_ref[0, hc] * jnp.float32(0.4))
        lm = jnp.where(mask, q, neg_inf)
        m = jnp.max(lm, axis=0, keepdims=True)
        a = jnp.where(mask, jnp.exp(q - m), 0.0)
        xl_h1 = jnp.concatenate([xl_h, ones_col], axis=1)
        oh = jax.lax.dot_general(a, xl_h1, (((0,), (0,)), ((), ())),
                                 preferred_element_type=jnp.float32)
        denom = oh[:, C:C + 1]
        head_outs.append(oh[:, :C] / jnp.where(denom > 0, denom, 1.0))
    y = jnp.concatenate(head_outs, axis=1) + bias_ref[0]
    mean = jnp.mean(y, axis=1, keepdims=True)
    yc = y - mean
    var = jnp.mean(yc * yc, axis=1, keepdims=True)
    out_ref[0] = yc * jax.lax.rsqrt(var + 1e-5) * gamma_ref[0] + beta_ref[0]


@jax.jit
def kernel(x, adj, Wl, bl, Wr, br, att, bias, gamma, beta):
    row_spec = pl.BlockSpec((1, HEADS * C), lambda b, jt: (0, 0))
    out = pl.pallas_call(
        _gat_tile_kernel,
        grid=(B, NJ),
        in_specs=[
            pl.BlockSpec((1, N, IN_DIM), lambda b, jt: (b, 0, 0)),
            pl.BlockSpec((1, N, TJ), lambda b, jt: (b, 0, jt)),
            pl.BlockSpec((IN_DIM, HEADS * C), lambda b, jt: (0, 0)),
            row_spec,
            pl.BlockSpec((IN_DIM, HEADS * C), lambda b, jt: (0, 0)),
            row_spec,
            row_spec,
            pl.BlockSpec((HEADS * C, 1), lambda b, jt: (0, 0)),
            row_spec,
            row_spec,
            row_spec,
        ],
        out_specs=pl.BlockSpec((1, TJ, OUT_DIM), lambda b, jt: (b, jt, 0)),
        out_shape=jax.ShapeDtypeStruct((B, N, OUT_DIM), jnp.float32),
        compiler_params=pltpu.CompilerParams(
            dimension_semantics=("parallel", "parallel")),
    )(x, adj, Wl, bl.reshape(1, -1), Wr, br.reshape(1, -1),
      att.reshape(1, -1), att.reshape(-1, 1), bias.reshape(1, -1),
      gamma.reshape(1, -1), beta.reshape(1, -1))
    return out


# speedup vs baseline: 1.6598x; 1.6598x over previous
import jax
import jax.numpy as jnp
from jax.experimental import pallas as pl
from jax.experimental.pallas import tpu as pltpu

B, N, IN_DIM, OUT_DIM, HEADS = 4, 512, 128, 128, 4
C = OUT_DIM // HEADS
NEG_SLOPE = 0.2
TJ = N


def _gat_tile_kernel(x_ref, adj_ref, wl_ref, bl_ref, wr_ref, br_ref,
                     att_ref, att_col_ref, bias_ref, gamma_ref, beta_ref,
                     out_ref):
    xb = x_ref[0]
    xl = jnp.dot(xb, wl_ref[...], preferred_element_type=jnp.float32)
    xl = xl + bl_ref[0]
    xrt = jax.lax.dot_general(wr_ref[...], xb, (((0,), (1,)), ((), ())),
                              preferred_element_type=jnp.float32)
    mask = adj_ref[0] != 0
    ones_col = jnp.ones((N, 1), jnp.float32)
    head_outs = []
    for h in range(HEADS):
        xl_h = xl[:, h * C:(h + 1) * C]
        u = jnp.dot(xl_h, att_col_ref[h * C:(h + 1) * C, :],
                    preferred_element_type=jnp.float32)
        q = u * jnp.float32(0.6)
        for c in range(C):
            hc = h * C + c
            col = xl_h[:, c:c + 1] + br_ref[0, hc]
            z = col + xrt[hc:hc + 1, :]
            q = q + jnp.abs(z) * (att_ref[0, hc] * jnp.float32(0.4))
        m = jnp.max(q, axis=0, keepdims=True)
        a = jnp.where(mask, jnp.exp(q - m), 0.0)
        xl_h1 = jnp.concatenate([xl_h, ones_col], axis=1)
        oh = jax.lax.dot_general(a, xl_h1, (((0,), (0,)), ((), ())),
                                 preferred_element_type=jnp.float32)
        denom = oh[:, C:C + 1]
        head_outs.append(oh[:, :C] / jnp.where(denom > 0, denom, 1.0))
    y = jnp.concatenate(head_outs, axis=1) + bias_ref[0]
    mean = jnp.mean(y, axis=1, keepdims=True)
    yc = y - mean
    var = jnp.mean(yc * yc, axis=1, keepdims=True)
    out_ref[0] = yc * jax.lax.rsqrt(var + 1e-5) * gamma_ref[0] + beta_ref[0]


@jax.jit
def kernel(x, adj, Wl, bl, Wr, br, att, bias, gamma, beta):
    row_spec = pl.BlockSpec((1, HEADS * C), lambda b: (0, 0))
    out = pl.pallas_call(
        _gat_tile_kernel,
        grid=(B,),
        in_specs=[
            pl.BlockSpec((1, N, IN_DIM), lambda b: (b, 0, 0)),
            pl.BlockSpec((1, N, N), lambda b: (b, 0, 0)),
            pl.BlockSpec((IN_DIM, HEADS * C), lambda b: (0, 0)),
            row_spec,
            pl.BlockSpec((IN_DIM, HEADS * C), lambda b: (0, 0)),
            row_spec,
            row_spec,
            pl.BlockSpec((HEADS * C, 1), lambda b: (0, 0)),
            row_spec,
            row_spec,
            row_spec,
        ],
        out_specs=pl.BlockSpec((1, N, OUT_DIM), lambda b: (b, 0, 0)),
        out_shape=jax.ShapeDtypeStruct((B, N, OUT_DIM), jnp.float32),
        compiler_params=pltpu.CompilerParams(
            dimension_semantics=("parallel",)),
    )(x, adj, Wl, bl.reshape(1, -1), Wr, br.reshape(1, -1),
      att.reshape(1, -1), att.reshape(-1, 1), bias.reshape(1, -1),
      gamma.reshape(1, -1), beta.reshape(1, -1))
    return out
